# baseline (speedup 1.0000x reference)
import functools

import jax
import jax.numpy as jnp
from jax import lax
from jax.experimental import pallas as pl
from jax.experimental.pallas import tpu as pltpu
from jax.experimental.pallas import tpu_sc as plsc

NC = 2
NS = 16
NW = NC * NS
CHUNK = 80

f32 = jnp.float32
NA_W = 20


def _sc_mesh():
    return plsc.VectorSubcoreMesh(core_axis_name="c", subcore_axis_name="s")


_SC_PARAMS = pltpu.CompilerParams(use_tc_tiling_on_sc=False)


@functools.lru_cache(maxsize=None)
def _make_sc_pass(N, E, F, mode):
    two = mode == "na"
    R = 3
    assert N % NS == 0
    assert E % (NW * CHUNK) == 0
    nch = E // (NW * CHUNK)
    assert nch > 2 * R
    rpt = N // NS
    out_shape = (NC, 2, NS, rpt, F) if two else (NC, NS, rpt, F)

    scratch = (
        [pltpu.VMEM((nch, CHUNK), jnp.int32)] * 2
        + [pltpu.VMEM((CHUNK, F), f32)] * R
        + [pltpu.SemaphoreType.DMA] * (2 * R)
        + [pltpu.VMEM_SHARED((N, F), f32)] * (2 if two else 1)
    )

    @functools.partial(
        pl.kernel,
        mesh=_sc_mesh(),
        out_type=jax.ShapeDtypeStruct(out_shape, f32),
        scratch_types=scratch,
        compiler_params=_SC_PARAMS,
    )
    def k(data_hbm, src_hbm, dst_hbm, z_hbm, out_hbm, *s):
        idxs, idxd = s[0], s[1]
        rows = s[2:2 + R]
        semg = s[2 + R:2 + 2 * R]
        sems = s[2 + 2 * R:2 + 3 * R]
        accs = s[2 + 3 * R:]
        cid = lax.axis_index("c")
        sid = lax.axis_index("s")
        wid = sid * NC + cid
        r0 = sid * rpt
        c0 = wid * nch

        for a in accs:
            pltpu.sync_copy(z_hbm.at[sid], a.at[pl.ds(r0, rpt)])
        pltpu.sync_copy(src_hbm.at[pl.ds(c0, nch)], idxs)
        pltpu.sync_copy(dst_hbm.at[pl.ds(c0, nch)], idxd)
        plsc.subcore_barrier()

        def load(i, sl):
            if two:
                return pltpu.make_async_copy(
                    data_hbm.at[pl.ds((c0 + i) * CHUNK, CHUNK)],
                    rows[sl], semg[sl])
            return pltpu.make_async_copy(
                data_hbm.at[idxs.at[i]], rows[sl], semg[sl])

        def scat_pairs(i, sl):
            if two:
                return [(rows[sl], accs[0].at[idxs.at[i]]),
                        (rows[sl], accs[1].at[idxd.at[i]])]
            return [(rows[sl], accs[0].at[idxd.at[i]])]

        def scats_start(i, sl):
            for a, b in scat_pairs(i, sl):
                pltpu.async_copy(a, b, sems[sl], add=True)

        def scats_wait(i, sl):
            for a, b in scat_pairs(i, sl):
                pltpu.make_async_copy(a, b, sems[sl]).wait()

        def step(i, r):
            load(i, r).wait()
            scats_start(i, r)
            r1 = (r + 1) % R
            j = i + 1

            @pl.when(j < nch)
            def _():
                @pl.when(j >= R)
                def _():
                    scats_wait(j - R, r1)

                load(j, r1).start()

        load(0, 0).start()

        def body(g, carry):
            for r in range(R):
                step(g * R + r, r)
            return carry

        ngrp = nch // R
        lax.fori_loop(0, ngrp, body, 0)
        for q in range(nch % R):
            step(ngrp * R + q, q)
        for q in range(R):
            i = nch - R + q
            scats_wait(i, i % R)

        plsc.subcore_barrier()
        if two:
            pltpu.sync_copy(accs[0].at[pl.ds(r0, rpt)], out_hbm.at[cid, 0, sid])
            pltpu.sync_copy(accs[1].at[pl.ds(r0, rpt)], out_hbm.at[cid, 1, sid])
        else:
            pltpu.sync_copy(accs[0].at[pl.ds(r0, rpt)], out_hbm.at[cid, sid])

    return k


BN = 2000


def _tc1(accs, x, w1a, w1b):
    N, DF = x.shape
    H = w1a.shape[1]
    grid = (N // BN,)

    def body(a_ref, x_ref, wa_ref, wb_ref, g_ref, dis_ref):
        a = a_ref[...]
        attr = a[0, 0, :, 0:16] + a[0, 1, :, 0:16] + a[1, 0, :, 0:16] + a[1, 1, :, 0:16]
        cnt = a[0, 0, :, 16:17] + a[0, 1, :, 16:17] + a[1, 0, :, 16:17] + a[1, 1, :, 16:17]
        degd = a[0, 1, :, 16:17] + a[1, 1, :, 16:17]
        na = attr / (cnt + 1e-8)
        dis = lax.rsqrt(degd + 1.0)
        h = (jnp.dot(x_ref[...], wa_ref[...], preferred_element_type=f32)
             + jnp.dot(na, wb_ref[...], preferred_element_type=f32))
        g_ref[...] = h * dis
        dis_ref[...] = dis

    return pl.pallas_call(
        body,
        grid=grid,
        in_specs=[
            pl.BlockSpec((NC, 2, BN, NA_W), lambda i: (0, 0, i, 0)),
            pl.BlockSpec((BN, DF), lambda i: (i, 0)),
            pl.BlockSpec((DF, H), lambda i: (0, 0)),
            pl.BlockSpec((16, H), lambda i: (0, 0)),
        ],
        out_specs=[
            pl.BlockSpec((BN, H), lambda i: (i, 0)),
            pl.BlockSpec((BN, 1), lambda i: (i, 0)),
        ],
        out_shape=[
            jax.ShapeDtypeStruct((N, H), f32),
            jax.ShapeDtypeStruct((N, 1), f32),
        ],
    )(accs, x, w1a, w1b)


def _tc_mid(aggs, g, dis, b, w):
    N, F = g.shape
    Fo = w.shape[1]
    grid = (N // BN,)

    def body(a_ref, g_ref, dis_ref, b_ref, w_ref, out_ref):
        a = a_ref[...]
        dis = dis_ref[...]
        s = dis * (a[0] + a[1] + g_ref[...]) + b_ref[...]
        h = jnp.maximum(s, 0.0)
        out_ref[...] = jnp.dot(h, w_ref[...], preferred_element_type=f32) * dis

    return pl.pallas_call(
        body,
        grid=grid,
        in_specs=[
            pl.BlockSpec((NC, BN, F), lambda i: (0, i, 0)),
            pl.BlockSpec((BN, F), lambda i: (i, 0)),
            pl.BlockSpec((BN, 1), lambda i: (i, 0)),
            pl.BlockSpec((1, F), lambda i: (0, 0)),
            pl.BlockSpec((F, Fo), lambda i: (0, 0)),
        ],
        out_specs=pl.BlockSpec((BN, Fo), lambda i: (i, 0)),
        out_shape=jax.ShapeDtypeStruct((N, Fo), f32),
    )(aggs, g, dis, b, w)


def _tc_relu_scale(aggs, g, dis, b):
    N, F = g.shape
    grid = (N // BN,)

    def body(a_ref, g_ref, dis_ref, b_ref, out_ref):
        a = a_ref[...]
        dis = dis_ref[...]
        z = jnp.maximum(dis * (a[0] + a[1] + g_ref[...]) + b_ref[...], 0.0)
        out_ref[...] = z * dis

    return pl.pallas_call(
        body,
        grid=grid,
        in_specs=[
            pl.BlockSpec((NC, BN, F), lambda i: (0, i, 0)),
            pl.BlockSpec((BN, F), lambda i: (i, 0)),
            pl.BlockSpec((BN, 1), lambda i: (i, 0)),
            pl.BlockSpec((1, F), lambda i: (0, 0)),
        ],
        out_specs=pl.BlockSpec((BN, F), lambda i: (i, 0)),
        out_shape=jax.ShapeDtypeStruct((N, F), f32),
    )(aggs, g, dis, b)


def _tc_mm_relu_scale(aggs, g, dis, b, w):
    N, F = g.shape
    Fo = w.shape[1]
    grid = (N // BN,)

    def body(a_ref, g_ref, dis_ref, b_ref, w_ref, out_ref):
        a = a_ref[...]
        dis = dis_ref[...]
        t = jnp.dot(a[0] + a[1] + g_ref[...], w_ref[...],
                    preferred_element_type=f32)
        h = jnp.maximum(dis * t + b_ref[...], 0.0)
        out_ref[...] = h * dis

    return pl.pallas_call(
        body,
        grid=grid,
        in_specs=[
            pl.BlockSpec((NC, BN, F), lambda i: (0, i, 0)),
            pl.BlockSpec((BN, F), lambda i: (i, 0)),
            pl.BlockSpec((BN, 1), lambda i: (i, 0)),
            pl.BlockSpec((1, Fo), lambda i: (0, 0)),
            pl.BlockSpec((F, Fo), lambda i: (0, 0)),
        ],
        out_specs=pl.BlockSpec((BN, Fo), lambda i: (i, 0)),
        out_shape=jax.ShapeDtypeStruct((N, Fo), f32),
    )(aggs, g, dis, b, w)


def _tc_fin(aggs, g, dis, b, w):
    N, F = g.shape
    Fo = w.shape[1]
    grid = (N // BN,)

    def body(a_ref, g_ref, dis_ref, b_ref, w_ref, out_ref):
        a = a_ref[...]
        t = jnp.dot(a[0] + a[1] + g_ref[...], w_ref[...],
                    preferred_element_type=f32)
        out_ref[...] = dis_ref[...] * t + b_ref[...]

    return pl.pallas_call(
        body,
        grid=grid,
        in_specs=[
            pl.BlockSpec((NC, BN, F), lambda i: (0, i, 0)),
            pl.BlockSpec((BN, F), lambda i: (i, 0)),
            pl.BlockSpec((BN, 1), lambda i: (i, 0)),
            pl.BlockSpec((1, Fo), lambda i: (0, 0)),
            pl.BlockSpec((F, Fo), lambda i: (0, 0)),
        ],
        out_specs=pl.BlockSpec((BN, Fo), lambda i: (i, 0)),
        out_shape=jax.ShapeDtypeStruct((N, Fo), f32),
    )(aggs, g, dis, b, w)


def kernel(x, edge_index, edge_attr, W1, b1, W2, b2, Wd1, bd1, Wd2, bd2):
    N, DF = x.shape
    E = edge_index.shape[1]
    DE = edge_attr.shape[1]
    src = edge_index[0]
    dst = edge_index[1]

    ea_plus = jnp.concatenate(
        [edge_attr, jnp.ones((E, 1), f32),
         jnp.zeros((E, NA_W - DE - 1), f32)], axis=1)

    rpt = N // NS
    src2d = src.reshape(E // CHUNK, CHUNK)
    dst2d = dst.reshape(E // CHUNK, CHUNK)
    na_acc = _make_sc_pass(N, E, NA_W, "na")(
        ea_plus, src2d, dst2d, jnp.zeros((NS, rpt, NA_W), f32)
    ).reshape(NC, 2, N, NA_W)
    g1, dis = _tc1(na_acc, x, W1[:DF], W1[DF:])

    def conv_agg(g):
        F = g.shape[1]
        return _make_sc_pass(N, E, F, "agg")(
            g, src2d, dst2d, jnp.zeros((NS, rpt, F), f32)
        ).reshape(NC, N, F)

    g2 = _tc_mid(conv_agg(g1), g1, dis, b1.reshape(1, -1), W2)
    zd = _tc_relu_scale(conv_agg(g2), g2, dis, b2.reshape(1, -1))
    dhd = _tc_mm_relu_scale(conv_agg(zd), zd, dis, bd1.reshape(1, -1), Wd1)
    return _tc_fin(conv_agg(dhd), dhd, dis, bd2.reshape(1, -1), Wd2)


# speedup vs baseline: 1.1236x; 1.1236x over previous
import functools

import jax
import jax.numpy as jnp
from jax import lax
from jax.experimental import pallas as pl
from jax.experimental.pallas import tpu as pltpu
from jax.experimental.pallas import tpu_sc as plsc

NC = 2
NS = 16
NW = NC * NS
CHUNK = 80

f32 = jnp.float32
NA_W = 32


def _sc_mesh():
    return plsc.VectorSubcoreMesh(core_axis_name="c", subcore_axis_name="s")


_SC_PARAMS = pltpu.CompilerParams(use_tc_tiling_on_sc=False)


@functools.lru_cache(maxsize=None)
def _make_sc_pass(N, E, F, mode):
    two = mode == "na"
    R = 3
    RI = 2 * R
    assert N % NS == 0
    assert E % (NW * CHUNK) == 0
    nch = E // (NW * CHUNK)
    assert nch > 2 * RI
    rpt = N // NS
    out_shape = (NC, 2, NS, rpt, F) if two else (NC, NS, rpt, F)
    tiled = (not two) and F % 128 == 0

    scratch = (
        [pltpu.VMEM((CHUNK,), jnp.int32)] * (2 * RI)
        + [pltpu.VMEM((CHUNK, F), f32)] * R
        + [pltpu.SemaphoreType.DMA] * (RI + 2 * R)
        + [pltpu.VMEM_SHARED((N, F), f32)] * (2 if two else 1)
    )

    @functools.partial(
        pl.kernel,
        mesh=_sc_mesh(),
        out_type=jax.ShapeDtypeStruct(out_shape, f32),
        scratch_types=scratch,
        compiler_params=None if tiled else _SC_PARAMS,
    )
    def k(data_hbm, src_hbm, dst_hbm, z_hbm, out_hbm, *s):
        idxs = s[0:RI]
        idxd = s[RI:2 * RI]
        rows = s[2 * RI:2 * RI + R]
        semi = s[2 * RI + R:3 * RI + R]
        semg = s[3 * RI + R:3 * RI + 2 * R]
        sems = s[3 * RI + 2 * R:3 * RI + 3 * R]
        accs = s[3 * RI + 3 * R:]
        cid = lax.axis_index("c")
        sid = lax.axis_index("s")
        wid = sid * NC + cid
        r0 = sid * rpt
        e0 = wid * nch * CHUNK

        for a in accs:
            pltpu.sync_copy(z_hbm.at[sid], a.at[pl.ds(r0, rpt)])
        plsc.subcore_barrier()

        def maybe(c, fn):
            if isinstance(c, bool):
                if c:
                    fn()
            else:
                pl.when(c)(fn)

        def idx_pair(i, si):
            sl_src = src_hbm.at[pl.ds(e0 + i * CHUNK, CHUNK)]
            sl_dst = dst_hbm.at[pl.ds(e0 + i * CHUNK, CHUNK)]
            return [(sl_src, idxs[si]), (sl_dst, idxd[si])]

        def idx_issue(i, si):
            for a, b in idx_pair(i, si):
                pltpu.async_copy(a, b, semi[si])

        def idx_wait(i, si):
            for a, b in idx_pair(i, si):
                pltpu.make_async_copy(a, b, semi[si]).wait()

        def load(i, sl, si):
            if two:
                return pltpu.make_async_copy(
                    data_hbm.at[pl.ds(e0 + i * CHUNK, CHUNK)],
                    rows[sl], semg[sl])
            return pltpu.make_async_copy(
                data_hbm.at[idxs[si]], rows[sl], semg[sl])

        def scat_pairs(i, sl, si):
            if two:
                return [(rows[sl], accs[0].at[idxs[si]]),
                        (rows[sl], accs[1].at[idxd[si]])]
            return [(rows[sl], accs[0].at[idxd[si]])]

        def scats_start(i, sl, si):
            for a, b in scat_pairs(i, sl, si):
                pltpu.async_copy(a, b, sems[sl], add=True)

        def scats_wait(i, sl, si):
            for a, b in scat_pairs(i, sl, si):
                pltpu.make_async_copy(a, b, sems[sl]).wait()

        def step(i, si):
            sl = si % R
            load(i, sl, si).wait()
            scats_start(i, sl, si)
            j = i + 1
            si1 = (si + 1) % RI
            sl1 = si1 % R

            def cont():
                maybe(j >= R,
                      lambda: scats_wait(j - R, sl1, (si1 - R) % RI))
                maybe(i + R < nch, lambda: idx_issue(i + R, (si + R) % RI))
                idx_wait(j, si1)
                load(j, sl1, si1).start()

            maybe(j < nch, cont)

        for q in range(R):
            idx_issue(q, q)
        idx_wait(0, 0)
        load(0, 0, 0).start()

        ngrp = nch // RI

        def body(g, carry):
            for r in range(RI):
                step(g * RI + r, r)
            return carry

        lax.fori_loop(0, ngrp, body, 0)
        for q in range(nch % RI):
            step(ngrp * RI + q, q)
        for q in range(R):
            i = nch - R + q
            scats_wait(i, i % R, i % RI)

        plsc.subcore_barrier()
        if two:
            pltpu.sync_copy(accs[0].at[pl.ds(r0, rpt)], out_hbm.at[cid, 0, sid])
            pltpu.sync_copy(accs[1].at[pl.ds(r0, rpt)], out_hbm.at[cid, 1, sid])
        else:
            pltpu.sync_copy(accs[0].at[pl.ds(r0, rpt)], out_hbm.at[cid, sid])

    return k


BN = 2000


def _tc1(accs, x, w1a, w1b):
    N, DF = x.shape
    H = w1a.shape[1]
    grid = (N // BN,)

    def body(a_ref, x_ref, wa_ref, wb_ref, g_ref, dis_ref):
        a = a_ref[...]
        attr = a[0, 0, :, 0:16] + a[0, 1, :, 0:16] + a[1, 0, :, 0:16] + a[1, 1, :, 0:16]
        cnt = a[0, 0, :, 16:17] + a[0, 1, :, 16:17] + a[1, 0, :, 16:17] + a[1, 1, :, 16:17]
        degd = a[0, 1, :, 16:17] + a[1, 1, :, 16:17]
        na = attr / (cnt + 1e-8)
        dis = lax.rsqrt(degd + 1.0)
        h = (jnp.dot(x_ref[...], wa_ref[...], preferred_element_type=f32)
             + jnp.dot(na, wb_ref[...], preferred_element_type=f32))
        g_ref[...] = h * dis
        dis_ref[...] = dis

    return pl.pallas_call(
        body,
        grid=grid,
        in_specs=[
            pl.BlockSpec((NC, 2, BN, NA_W), lambda i: (0, 0, i, 0)),
            pl.BlockSpec((BN, DF), lambda i: (i, 0)),
            pl.BlockSpec((DF, H), lambda i: (0, 0)),
            pl.BlockSpec((16, H), lambda i: (0, 0)),
        ],
        out_specs=[
            pl.BlockSpec((BN, H), lambda i: (i, 0)),
            pl.BlockSpec((BN, 1), lambda i: (i, 0)),
        ],
        out_shape=[
            jax.ShapeDtypeStruct((N, H), f32),
            jax.ShapeDtypeStruct((N, 1), f32),
        ],
    )(accs, x, w1a, w1b)


def _tc_mid(aggs, g, dis, b, w):
    N, F = g.shape
    Fo = w.shape[1]
    grid = (N // BN,)

    def body(a_ref, g_ref, dis_ref, b_ref, w_ref, out_ref):
        a = a_ref[...]
        dis = dis_ref[...]
        s = dis * (a[0] + a[1] + g_ref[...]) + b_ref[...]
        h = jnp.maximum(s, 0.0)
        out_ref[...] = jnp.dot(h, w_ref[...], preferred_element_type=f32) * dis

    return pl.pallas_call(
        body,
        grid=grid,
        in_specs=[
            pl.BlockSpec((NC, BN, F), lambda i: (0, i, 0)),
            pl.BlockSpec((BN, F), lambda i: (i, 0)),
            pl.BlockSpec((BN, 1), lambda i: (i, 0)),
            pl.BlockSpec((1, F), lambda i: (0, 0)),
            pl.BlockSpec((F, Fo), lambda i: (0, 0)),
        ],
        out_specs=pl.BlockSpec((BN, Fo), lambda i: (i, 0)),
        out_shape=jax.ShapeDtypeStruct((N, Fo), f32),
    )(aggs, g, dis, b, w)


def _tc_relu_scale(aggs, g, dis, b):
    N, F = g.shape
    grid = (N // BN,)

    def body(a_ref, g_ref, dis_ref, b_ref, out_ref):
        a = a_ref[...]
        dis = dis_ref[...]
        z = jnp.maximum(dis * (a[0] + a[1] + g_ref[...]) + b_ref[...], 0.0)
        out_ref[...] = z * dis

    return pl.pallas_call(
        body,
        grid=grid,
        in_specs=[
            pl.BlockSpec((NC, BN, F), lambda i: (0, i, 0)),
            pl.BlockSpec((BN, F), lambda i: (i, 0)),
            pl.BlockSpec((BN, 1), lambda i: (i, 0)),
            pl.BlockSpec((1, F), lambda i: (0, 0)),
        ],
        out_specs=pl.BlockSpec((BN, F), lambda i: (i, 0)),
        out_shape=jax.ShapeDtypeStruct((N, F), f32),
    )(aggs, g, dis, b)


def _tc_mm_relu_scale(aggs, g, dis, b, w):
    N, F = g.shape
    Fo = w.shape[1]
    grid = (N // BN,)

    def body(a_ref, g_ref, dis_ref, b_ref, w_ref, out_ref):
        a = a_ref[...]
        dis = dis_ref[...]
        t = jnp.dot(a[0] + a[1] + g_ref[...], w_ref[...],
                    preferred_element_type=f32)
        h = jnp.maximum(dis * t + b_ref[...], 0.0)
        out_ref[...] = h * dis

    return pl.pallas_call(
        body,
        grid=grid,
        in_specs=[
            pl.BlockSpec((NC, BN, F), lambda i: (0, i, 0)),
            pl.BlockSpec((BN, F), lambda i: (i, 0)),
            pl.BlockSpec((BN, 1), lambda i: (i, 0)),
            pl.BlockSpec((1, Fo), lambda i: (0, 0)),
            pl.BlockSpec((F, Fo), lambda i: (0, 0)),
        ],
        out_specs=pl.BlockSpec((BN, Fo), lambda i: (i, 0)),
        out_shape=jax.ShapeDtypeStruct((N, Fo), f32),
    )(aggs, g, dis, b, w)


def _tc_fin(aggs, g, dis, b, w):
    N, F = g.shape
    Fo = w.shape[1]
    grid = (N // BN,)

    def body(a_ref, g_ref, dis_ref, b_ref, w_ref, out_ref):
        a = a_ref[...]
        t = jnp.dot(a[0] + a[1] + g_ref[...], w_ref[...],
                    preferred_element_type=f32)
        out_ref[...] = dis_ref[...] * t + b_ref[...]

    return pl.pallas_call(
        body,
        grid=grid,
        in_specs=[
            pl.BlockSpec((NC, BN, F), lambda i: (0, i, 0)),
            pl.BlockSpec((BN, F), lambda i: (i, 0)),
            pl.BlockSpec((BN, 1), lambda i: (i, 0)),
            pl.BlockSpec((1, Fo), lambda i: (0, 0)),
            pl.BlockSpec((F, Fo), lambda i: (0, 0)),
        ],
        out_specs=pl.BlockSpec((BN, Fo), lambda i: (i, 0)),
        out_shape=jax.ShapeDtypeStruct((N, Fo), f32),
    )(aggs, g, dis, b, w)


def kernel(x, edge_index, edge_attr, W1, b1, W2, b2, Wd1, bd1, Wd2, bd2):
    N, DF = x.shape
    E = edge_index.shape[1]
    DE = edge_attr.shape[1]
    src = edge_index[0]
    dst = edge_index[1]

    ea_plus = jnp.concatenate(
        [edge_attr, jnp.ones((E, 1), f32),
         jnp.zeros((E, NA_W - DE - 1), f32)], axis=1)

    rpt = N // NS

    na_acc = _make_sc_pass(N, E, NA_W, "na")(
        ea_plus, src, dst, jnp.zeros((NS, rpt, NA_W), f32)
    ).reshape(NC, 2, N, NA_W)
    g1, dis = _tc1(na_acc, x, W1[:DF], W1[DF:])

    def conv_agg(g):
        F = g.shape[1]
        return _make_sc_pass(N, E, F, "agg")(
            g, src, dst, jnp.zeros((NS, rpt, F), f32)
        ).reshape(NC, N, F)

    g2 = _tc_mid(conv_agg(g1), g1, dis, b1.reshape(1, -1), W2)
    zd = _tc_relu_scale(conv_agg(g2), g2, dis, b2.reshape(1, -1))
    dhd = _tc_mm_relu_scale(conv_agg(zd), zd, dis, bd1.reshape(1, -1), Wd1)
    return _tc_fin(conv_agg(dhd), dhd, dis, bd2.reshape(1, -1), Wd2)


# speedup vs baseline: 1.1441x; 1.0182x over previous
import functools

import jax
import jax.numpy as jnp
from jax import lax
from jax.experimental import pallas as pl
from jax.experimental.pallas import tpu as pltpu
from jax.experimental.pallas import tpu_sc as plsc

NC = 2
NS = 16
NW = NC * NS
CHUNK = 80

f32 = jnp.float32
NA_W = 32


def _sc_mesh():
    return plsc.VectorSubcoreMesh(core_axis_name="c", subcore_axis_name="s")


_SC_PARAMS = pltpu.CompilerParams(use_tc_tiling_on_sc=False)


@functools.lru_cache(maxsize=None)
def _make_sc_pass(N, E, F, mode):
    two = mode == "na"
    R = 3
    RI = 2 * R
    assert N % NS == 0
    assert E % (NW * CHUNK) == 0
    nch = E // (NW * CHUNK)
    assert nch > 2 * RI
    rpt = N // NS
    out_shape = (NC, 2, NS, rpt, F) if two else (NC, NS, rpt, F)
    tiled = False

    scratch = (
        [pltpu.VMEM((CHUNK,), jnp.int32)] * (2 * RI)
        + [pltpu.VMEM((CHUNK, F), f32)] * R
        + [pltpu.SemaphoreType.DMA] * (RI + 2 * R)
        + [pltpu.VMEM_SHARED((N, F), f32)] * (2 if two else 1)
    )

    @functools.partial(
        pl.kernel,
        mesh=_sc_mesh(),
        out_type=jax.ShapeDtypeStruct(out_shape, f32),
        scratch_types=scratch,
        compiler_params=None if tiled else _SC_PARAMS,
    )
    def k(data_hbm, src_hbm, dst_hbm, z_hbm, out_hbm, *s):
        idxs = s[0:RI]
        idxd = s[RI:2 * RI]
        rows = s[2 * RI:2 * RI + R]
        semi = s[2 * RI + R:3 * RI + R]
        semg = s[3 * RI + R:3 * RI + 2 * R]
        sems = s[3 * RI + 2 * R:3 * RI + 3 * R]
        accs = s[3 * RI + 3 * R:]
        cid = lax.axis_index("c")
        sid = lax.axis_index("s")
        wid = sid * NC + cid
        r0 = sid * rpt
        e0 = wid * nch * CHUNK

        for a in accs:
            pltpu.sync_copy(z_hbm.at[sid], a.at[pl.ds(r0, rpt)])
        plsc.subcore_barrier()

        def maybe(c, fn):
            if isinstance(c, bool):
                if c:
                    fn()
            else:
                pl.when(c)(fn)

        def idx_pair(i, si):
            sl_src = src_hbm.at[pl.ds(e0 + i * CHUNK, CHUNK)]
            sl_dst = dst_hbm.at[pl.ds(e0 + i * CHUNK, CHUNK)]
            return [(sl_src, idxs[si]), (sl_dst, idxd[si])]

        def idx_issue(i, si):
            for a, b in idx_pair(i, si):
                pltpu.async_copy(a, b, semi[si])

        def idx_wait(i, si):
            for a, b in idx_pair(i, si):
                pltpu.make_async_copy(a, b, semi[si]).wait()

        def load(i, sl, si):
            if two:
                return pltpu.make_async_copy(
                    data_hbm.at[pl.ds(e0 + i * CHUNK, CHUNK)],
                    rows[sl], semg[sl])
            return pltpu.make_async_copy(
                data_hbm.at[idxs[si]], rows[sl], semg[sl])

        def scat_pairs(i, sl, si):
            if two:
                return [(rows[sl], accs[0].at[idxs[si]]),
                        (rows[sl], accs[1].at[idxd[si]])]
            return [(rows[sl], accs[0].at[idxd[si]])]

        def scats_start(i, sl, si):
            for a, b in scat_pairs(i, sl, si):
                pltpu.async_copy(a, b, sems[sl], add=True)

        def scats_wait(i, sl, si):
            for a, b in scat_pairs(i, sl, si):
                pltpu.make_async_copy(a, b, sems[sl]).wait()

        def step(i, si):
            sl = si % R
            load(i, sl, si).wait()
            scats_start(i, sl, si)
            j = i + 1
            si1 = (si + 1) % RI
            sl1 = si1 % R

            def cont():
                maybe(j >= R,
                      lambda: scats_wait(j - R, sl1, (si1 - R) % RI))
                maybe(i + R < nch, lambda: idx_issue(i + R, (si + R) % RI))
                idx_wait(j, si1)
                load(j, sl1, si1).start()

            maybe(j < nch, cont)

        for q in range(R):
            idx_issue(q, q)
        idx_wait(0, 0)
        load(0, 0, 0).start()

        ngrp = nch // RI

        def body(g, carry):
            for r in range(RI):
                step(g * RI + r, r)
            return carry

        lax.fori_loop(0, ngrp, body, 0)
        for q in range(nch % RI):
            step(ngrp * RI + q, q)
        for q in range(R):
            i = nch - R + q
            scats_wait(i, i % R, i % RI)

        plsc.subcore_barrier()
        if two:
            pltpu.sync_copy(accs[0].at[pl.ds(r0, rpt)], out_hbm.at[cid, 0, sid])
            pltpu.sync_copy(accs[1].at[pl.ds(r0, rpt)], out_hbm.at[cid, 1, sid])
        else:
            pltpu.sync_copy(accs[0].at[pl.ds(r0, rpt)], out_hbm.at[cid, sid])

    return k


BN = 2000


def _tc1(accs, x, w1a, w1b):
    N, DF = x.shape
    H = w1a.shape[1]
    grid = (N // BN,)

    def body(a_ref, x_ref, wa_ref, wb_ref, g_ref, dis_ref):
        a = a_ref[...]
        attr = a[0, 0, :, 0:16] + a[0, 1, :, 0:16] + a[1, 0, :, 0:16] + a[1, 1, :, 0:16]
        cnt = a[0, 0, :, 16:17] + a[0, 1, :, 16:17] + a[1, 0, :, 16:17] + a[1, 1, :, 16:17]
        degd = a[0, 1, :, 16:17] + a[1, 1, :, 16:17]
        na = attr / (cnt + 1e-8)
        dis = lax.rsqrt(degd + 1.0)
        h = (jnp.dot(x_ref[...], wa_ref[...], preferred_element_type=f32)
             + jnp.dot(na, wb_ref[...], preferred_element_type=f32))
        g_ref[...] = h * dis
        dis_ref[...] = dis

    return pl.pallas_call(
        body,
        grid=grid,
        in_specs=[
            pl.BlockSpec((NC, 2, BN, NA_W), lambda i: (0, 0, i, 0)),
            pl.BlockSpec((BN, DF), lambda i: (i, 0)),
            pl.BlockSpec((DF, H), lambda i: (0, 0)),
            pl.BlockSpec((16, H), lambda i: (0, 0)),
        ],
        out_specs=[
            pl.BlockSpec((BN, H), lambda i: (i, 0)),
            pl.BlockSpec((BN, 1), lambda i: (i, 0)),
        ],
        out_shape=[
            jax.ShapeDtypeStruct((N, H), f32),
            jax.ShapeDtypeStruct((N, 1), f32),
        ],
    )(accs, x, w1a, w1b)


def _tc_mid(aggs, g, dis, b, w):
    N, F = g.shape
    Fo = w.shape[1]
    grid = (N // BN,)

    def body(a_ref, g_ref, dis_ref, b_ref, w_ref, out_ref):
        a = a_ref[...]
        dis = dis_ref[...]
        s = dis * (a[0] + a[1] + g_ref[...]) + b_ref[...]
        h = jnp.maximum(s, 0.0)
        out_ref[...] = jnp.dot(h, w_ref[...], preferred_element_type=f32) * dis

    return pl.pallas_call(
        body,
        grid=grid,
        in_specs=[
            pl.BlockSpec((NC, BN, F), lambda i: (0, i, 0)),
            pl.BlockSpec((BN, F), lambda i: (i, 0)),
            pl.BlockSpec((BN, 1), lambda i: (i, 0)),
            pl.BlockSpec((1, F), lambda i: (0, 0)),
            pl.BlockSpec((F, Fo), lambda i: (0, 0)),
        ],
        out_specs=pl.BlockSpec((BN, Fo), lambda i: (i, 0)),
        out_shape=jax.ShapeDtypeStruct((N, Fo), f32),
    )(aggs, g, dis, b, w)


def _tc_relu_scale(aggs, g, dis, b):
    N, F = g.shape
    grid = (N // BN,)

    def body(a_ref, g_ref, dis_ref, b_ref, out_ref):
        a = a_ref[...]
        dis = dis_ref[...]
        z = jnp.maximum(dis * (a[0] + a[1] + g_ref[...]) + b_ref[...], 0.0)
        out_ref[...] = z * dis

    return pl.pallas_call(
        body,
        grid=grid,
        in_specs=[
            pl.BlockSpec((NC, BN, F), lambda i: (0, i, 0)),
            pl.BlockSpec((BN, F), lambda i: (i, 0)),
            pl.BlockSpec((BN, 1), lambda i: (i, 0)),
            pl.BlockSpec((1, F), lambda i: (0, 0)),
        ],
        out_specs=pl.BlockSpec((BN, F), lambda i: (i, 0)),
        out_shape=jax.ShapeDtypeStruct((N, F), f32),
    )(aggs, g, dis, b)


def _tc_mm_relu_scale(aggs, g, dis, b, w):
    N, F = g.shape
    Fo = w.shape[1]
    grid = (N // BN,)

    def body(a_ref, g_ref, dis_ref, b_ref, w_ref, out_ref):
        a = a_ref[...]
        dis = dis_ref[...]
        t = jnp.dot(a[0] + a[1] + g_ref[...], w_ref[...],
                    preferred_element_type=f32)
        h = jnp.maximum(dis * t + b_ref[...], 0.0)
        out_ref[...] = h * dis

    return pl.pallas_call(
        body,
        grid=grid,
        in_specs=[
            pl.BlockSpec((NC, BN, F), lambda i: (0, i, 0)),
            pl.BlockSpec((BN, F), lambda i: (i, 0)),
            pl.BlockSpec((BN, 1), lambda i: (i, 0)),
            pl.BlockSpec((1, Fo), lambda i: (0, 0)),
            pl.BlockSpec((F, Fo), lambda i: (0, 0)),
        ],
        out_specs=pl.BlockSpec((BN, Fo), lambda i: (i, 0)),
        out_shape=jax.ShapeDtypeStruct((N, Fo), f32),
    )(aggs, g, dis, b, w)


def _tc_fin(aggs, g, dis, b, w):
    N, F = g.shape
    Fo = w.shape[1]
    grid = (N // BN,)

    def body(a_ref, g_ref, dis_ref, b_ref, w_ref, out_ref):
        a = a_ref[...]
        t = jnp.dot(a[0] + a[1] + g_ref[...], w_ref[...],
                    preferred_element_type=f32)
        out_ref[...] = dis_ref[...] * t + b_ref[...]

    return pl.pallas_call(
        body,
        grid=grid,
        in_specs=[
            pl.BlockSpec((NC, BN, F), lambda i: (0, i, 0)),
            pl.BlockSpec((BN, F), lambda i: (i, 0)),
            pl.BlockSpec((BN, 1), lambda i: (i, 0)),
            pl.BlockSpec((1, Fo), lambda i: (0, 0)),
            pl.BlockSpec((F, Fo), lambda i: (0, 0)),
        ],
        out_specs=pl.BlockSpec((BN, Fo), lambda i: (i, 0)),
        out_shape=jax.ShapeDtypeStruct((N, Fo), f32),
    )(aggs, g, dis, b, w)


def kernel(x, edge_index, edge_attr, W1, b1, W2, b2, Wd1, bd1, Wd2, bd2):
    N, DF = x.shape
    E = edge_index.shape[1]
    DE = edge_attr.shape[1]
    src = edge_index[0]
    dst = edge_index[1]

    ea_plus = jnp.concatenate(
        [edge_attr, jnp.ones((E, 1), f32),
         jnp.zeros((E, NA_W - DE - 1), f32)], axis=1)

    rpt = N // NS

    na_acc = _make_sc_pass(N, E, NA_W, "na")(
        ea_plus, src, dst, jnp.zeros((NS, rpt, NA_W), f32)
    ).reshape(NC, 2, N, NA_W)
    g1, dis = _tc1(na_acc, x, W1[:DF], W1[DF:])

    def conv_agg(g):
        F = g.shape[1]
        return _make_sc_pass(N, E, F, "agg")(
            g, src, dst, jnp.zeros((NS, rpt, F), f32)
        ).reshape(NC, N, F)

    g2 = _tc_mid(conv_agg(g1), g1, dis, b1.reshape(1, -1), W2)
    zd = _tc_relu_scale(conv_agg(g2), g2, dis, b2.reshape(1, -1))
    dhd = _tc_mm_relu_scale(conv_agg(zd), zd, dis, bd1.reshape(1, -1), Wd1)
    return _tc_fin(conv_agg(dhd), dhd, dis, bd2.reshape(1, -1), Wd2)
